# baseline (speedup 1.0000x reference)
import functools

import jax
import jax.numpy as jnp
from jax import lax
from jax.experimental import pallas as pl
from jax.experimental.pallas import tpu as pltpu
from jax.experimental.pallas import tpu_sc as plsc

B = 8192
N_ATOMS = 1048576
MAX_ELEM = 94
NC = 2
NS = 16
L = 16
NW = NC * NS
APW = N_ATOMS // NW
VECS = APW // L
ROWS = 64
RPT = ROWS // NS
NCHUNK = 2
CATOMS = APW // NCHUNK
VPC = VECS // NCHUNK

_mesh = plsc.VectorSubcoreMesh(core_axis_name="c", subcore_axis_name="s")


@functools.partial(
    pl.kernel,
    out_type=[
        jax.ShapeDtypeStruct((NC, ROWS, 128), jnp.float32),
        jax.ShapeDtypeStruct((NC, ROWS, 128), jnp.float32),
    ],
    mesh=_mesh,
    compiler_params=pltpu.CompilerParams(needs_layout_passes=False),
    scratch_types=[
        pltpu.VMEM((APW,), jnp.int32),
        pltpu.VMEM((APW,), jnp.int32),
        pltpu.VMEM((128,), jnp.float32),
        pltpu.VMEM((ROWS, 128), jnp.float32),
        pltpu.VMEM((ROWS, 128), jnp.float32),
        pltpu.VMEM((ROWS,), jnp.int32),
        pltpu.VMEM((RPT, 128), jnp.float32),
        pltpu.VMEM_SHARED((ROWS, 128), jnp.float32),
        pltpu.VMEM_SHARED((ROWS, 128), jnp.float32),
    ] + [pltpu.SemaphoreType.DMA] * NCHUNK,
)
def _sc_hist(w_hbm, atoms_hbm, segs_hbm, out_sum, out_cnt,
             atoms_v, segs_v, w_v, acc_s, acc_c, rows_v, bounce,
             sh_s, sh_c, *sems):
    cid = lax.axis_index("c")
    sid = lax.axis_index("s")
    wid = sid * NC + cid
    base = wid * APW
    pltpu.sync_copy(w_hbm, w_v)
    descs = []
    for k in range(NCHUNK):
        off = k * CATOMS
        descs.append((
            pltpu.async_copy(atoms_hbm.at[pl.ds(base + off, CATOMS)],
                             atoms_v.at[pl.ds(off, CATOMS)], sems[k]),
            pltpu.async_copy(segs_hbm.at[pl.ds(base + off, CATOMS)],
                             segs_v.at[pl.ds(off, CATOMS)], sems[k]),
        ))

    zeros = jnp.zeros((L,), jnp.float32)
    iota = lax.iota(jnp.int32, L)

    @plsc.parallel_loop(0, ROWS, unroll=8)
    def _(r):
        for k in range(128 // L):
            acc_s[r, pl.ds(k * L, L)] = zeros
            acc_c[r, pl.ds(k * L, L)] = zeros

    @plsc.parallel_loop(0, ROWS // L, unroll=4)
    def _(r):
        rows_v[pl.ds(r * L, L)] = iota + r * L

    inext = jnp.minimum(iota + 1, L - 1)
    iprev = jnp.maximum(iota - 1, 0)
    lastmask = iota == L - 1
    firstmask = iota == 0

    for k in range(NCHUNK):
        for d in descs[k]:
            d.wait()

        @plsc.parallel_loop(k * VPC, (k + 1) * VPC, unroll=16)
        def _(i):
            a = atoms_v[pl.ds(i * L, L)]
            s = segs_v[pl.ds(i * L, L)]
            v = plsc.load_gather(w_v, [a])
            s_next = s.at[inext].get(mode="promise_in_bounds")
            m_end = lastmask | (s != s_next)
            c = plsc.cumsum(v)
            q = jnp.where(m_end, iota, -1)
            r = plsc.cummax(q)
            rp = jnp.where(firstmask, -1,
                           r.at[iprev].get(mode="promise_in_bounds"))
            cp = jnp.where(
                rp >= 0,
                c.at[jnp.maximum(rp, 0)].get(mode="promise_in_bounds"),
                0.0)
            s_hi = jax.lax.shift_right_logical(s, 7)
            s_lo = jax.lax.bitwise_and(s, 127)
            plsc.addupdate_scatter(acc_s, [s_hi, s_lo], c - cp, mask=m_end)
            plsc.addupdate_scatter(acc_c, [s_hi, s_lo],
                                   (iota - rp).astype(jnp.float32),
                                   mask=m_end)

    @pl.when(sid == 0)
    def _():
        pltpu.sync_copy(acc_s, sh_s)
        pltpu.sync_copy(acc_c, sh_c)

    plsc.subcore_barrier()

    @pl.when(sid != 0)
    def _():
        pltpu.sync_copy(acc_s, sh_s.at[rows_v], add=True)
        pltpu.sync_copy(acc_c, sh_c.at[rows_v], add=True)

    plsc.subcore_barrier()

    pltpu.sync_copy(sh_s.at[pl.ds(sid * RPT, RPT)], bounce)
    pltpu.sync_copy(bounce, out_sum.at[cid, pl.ds(sid * RPT, RPT)])
    pltpu.sync_copy(sh_c.at[pl.ds(sid * RPT, RPT)], bounce)
    pltpu.sync_copy(bounce, out_cnt.at[cid, pl.ds(sid * RPT, RPT)])


def _combine_body(ps_ref, pc_ref, out_ref):
    s = ps_ref[0] + ps_ref[1]
    c = pc_ref[0] + pc_ref[1]
    out_ref[...] = s / jnp.maximum(c, 1.0)


def kernel(weight, atomic_numbers, segment_ids):
    w = jnp.pad(weight.reshape(-1), (0, 128 - MAX_ELEM))
    ps, pc = _sc_hist(w, atomic_numbers, segment_ids)
    out = pl.pallas_call(
        _combine_body,
        out_shape=jax.ShapeDtypeStruct((ROWS, 128), jnp.float32),
    )(ps, pc)
    return out.reshape(-1)


# speedup vs baseline: 1.3864x; 1.3864x over previous
import functools

import jax
import jax.numpy as jnp
from jax import lax
from jax.experimental import pallas as pl
from jax.experimental.pallas import tpu as pltpu
from jax.experimental.pallas import tpu_sc as plsc

B = 8192
N_ATOMS = 1048576
MAX_ELEM = 94
NC = 2
NS = 16
L = 16
NW = NC * NS
APW = N_ATOMS // NW
VECS = APW // L
ROWS = 64
RPT = ROWS // NS
NCHUNK = 2
CATOMS = APW // NCHUNK
VPC = VECS // NCHUNK

_mesh = plsc.VectorSubcoreMesh(core_axis_name="c", subcore_axis_name="s")


@functools.partial(
    pl.kernel,
    out_type=[
        jax.ShapeDtypeStruct((NC, ROWS, 128), jnp.float32),
        jax.ShapeDtypeStruct((NC, ROWS, 128), jnp.float32),
    ],
    mesh=_mesh,
    compiler_params=pltpu.CompilerParams(needs_layout_passes=False),
    scratch_types=[
        pltpu.VMEM((APW,), jnp.int32),
        pltpu.VMEM((APW,), jnp.int32),
        pltpu.VMEM((128,), jnp.float32),
        pltpu.VMEM((ROWS, 128), jnp.float32),
        pltpu.VMEM((ROWS, 128), jnp.float32),
        pltpu.VMEM((ROWS,), jnp.int32),
        pltpu.VMEM((RPT, 128), jnp.float32),
        pltpu.VMEM_SHARED((ROWS, 128), jnp.float32),
        pltpu.VMEM_SHARED((ROWS, 128), jnp.float32),
    ] + [pltpu.SemaphoreType.DMA] * NCHUNK,
)
def _sc_hist(w_hbm, atoms_hbm, segs_hbm, out_sum, out_cnt,
             atoms_v, segs_v, w_v, acc_s, acc_c, rows_v, bounce,
             sh_s, sh_c, *sems):
    cid = lax.axis_index("c")
    sid = lax.axis_index("s")
    wid = sid * NC + cid
    base = wid * APW
    pltpu.sync_copy(w_hbm, w_v)
    descs = []
    for k in range(NCHUNK):
        off = k * CATOMS
        descs.append((
            pltpu.async_copy(atoms_hbm.at[pl.ds(base + off, CATOMS)],
                             atoms_v.at[pl.ds(off, CATOMS)], sems[k]),
            pltpu.async_copy(segs_hbm.at[pl.ds(base + off, CATOMS)],
                             segs_v.at[pl.ds(off, CATOMS)], sems[k]),
        ))

    zeros = jnp.zeros((L,), jnp.float32)
    iota = lax.iota(jnp.int32, L)

    @plsc.parallel_loop(0, ROWS, unroll=8)
    def _(r):
        for k in range(128 // L):
            acc_s[r, pl.ds(k * L, L)] = zeros
            acc_c[r, pl.ds(k * L, L)] = zeros

    @plsc.parallel_loop(0, ROWS // L, unroll=4)
    def _(r):
        rows_v[pl.ds(r * L, L)] = iota + r * L

    inext = jnp.minimum(iota + 1, L - 1)
    iprev = jnp.maximum(iota - 1, 0)
    lastmask = iota == L - 1
    firstmask = iota == 0

    for k in range(NCHUNK):
        for d in descs[k]:
            d.wait()

        @plsc.parallel_loop(k * VPC, (k + 1) * VPC, unroll=8)
        def _(i):
            a = atoms_v[pl.ds(i * L, L)]
            s = segs_v[pl.ds(i * L, L)]
            v = plsc.load_gather(w_v, [a])
            s_next = s.at[inext].get(mode="promise_in_bounds")
            m_end = lastmask | (s != s_next)
            c = plsc.cumsum(v)
            q = jnp.where(m_end, iota, -1)
            r = plsc.cummax(q)
            rp = jnp.where(firstmask, -1,
                           r.at[iprev].get(mode="promise_in_bounds"))
            cp = jnp.where(
                rp >= 0,
                c.at[jnp.maximum(rp, 0)].get(mode="promise_in_bounds"),
                0.0)
            s_hi = jax.lax.shift_right_logical(s, 7)
            s_lo = jax.lax.bitwise_and(s, 127)
            plsc.addupdate_scatter(acc_s, [s_hi, s_lo], c - cp, mask=m_end)
            plsc.addupdate_scatter(acc_c, [s_hi, s_lo],
                                   (iota - rp).astype(jnp.float32),
                                   mask=m_end)

    @pl.when(sid == 0)
    def _():
        pltpu.sync_copy(acc_s, sh_s)
        pltpu.sync_copy(acc_c, sh_c)

    plsc.subcore_barrier()

    @pl.when(sid != 0)
    def _():
        pltpu.sync_copy(acc_s, sh_s.at[rows_v], add=True)
        pltpu.sync_copy(acc_c, sh_c.at[rows_v], add=True)

    plsc.subcore_barrier()

    pltpu.sync_copy(sh_s.at[pl.ds(sid * RPT, RPT)], bounce)
    pltpu.sync_copy(bounce, out_sum.at[cid, pl.ds(sid * RPT, RPT)])
    pltpu.sync_copy(sh_c.at[pl.ds(sid * RPT, RPT)], bounce)
    pltpu.sync_copy(bounce, out_cnt.at[cid, pl.ds(sid * RPT, RPT)])


def _combine_body(ps_ref, pc_ref, out_ref):
    s = ps_ref[0] + ps_ref[1]
    c = pc_ref[0] + pc_ref[1]
    out_ref[...] = s / jnp.maximum(c, 1.0)


def kernel(weight, atomic_numbers, segment_ids):
    w = jnp.pad(weight.reshape(-1), (0, 128 - MAX_ELEM))
    ps, pc = _sc_hist(w, atomic_numbers, segment_ids)
    out = pl.pallas_call(
        _combine_body,
        out_shape=jax.ShapeDtypeStruct((ROWS, 128), jnp.float32),
    )(ps, pc)
    return out.reshape(-1)


# speedup vs baseline: 1.4376x; 1.0369x over previous
import functools

import jax
import jax.numpy as jnp
from jax import lax
from jax.experimental import pallas as pl
from jax.experimental.pallas import tpu as pltpu
from jax.experimental.pallas import tpu_sc as plsc

B = 8192
N_ATOMS = 1048576
MAX_ELEM = 94
NC = 2
NS = 16
L = 16
NW = NC * NS
APW = N_ATOMS // NW
VECS = APW // L
ROWS = 64
RPT = ROWS // NS
NCHUNK = 2
CATOMS = APW // NCHUNK
VPC = VECS // NCHUNK

_mesh = plsc.VectorSubcoreMesh(core_axis_name="c", subcore_axis_name="s")


@functools.partial(
    pl.kernel,
    out_type=[
        jax.ShapeDtypeStruct((NC, ROWS, 128), jnp.float32),
        jax.ShapeDtypeStruct((NC, ROWS, 128), jnp.float32),
    ],
    mesh=_mesh,
    compiler_params=pltpu.CompilerParams(needs_layout_passes=False),
    scratch_types=[
        pltpu.VMEM((APW,), jnp.int32),
        pltpu.VMEM((APW,), jnp.int32),
        pltpu.VMEM((128,), jnp.float32),
        pltpu.VMEM((ROWS, 128), jnp.float32),
        pltpu.VMEM((ROWS, 128), jnp.float32),
        pltpu.VMEM((ROWS,), jnp.int32),
        pltpu.VMEM((RPT, 128), jnp.float32),
        pltpu.VMEM_SHARED((ROWS, 128), jnp.float32),
        pltpu.VMEM_SHARED((ROWS, 128), jnp.float32),
    ] + [pltpu.SemaphoreType.DMA] * (NCHUNK + 1),
)
def _sc_hist(w_hbm, atoms_hbm, segs_hbm, out_sum, out_cnt,
             atoms_v, segs_v, w_v, acc_s, acc_c, rows_v, bounce,
             sh_s, sh_c, *sems):
    cid = lax.axis_index("c")
    sid = lax.axis_index("s")
    wid = sid * NC + cid
    base = wid * APW
    w_desc = pltpu.async_copy(w_hbm, w_v, sems[NCHUNK])
    descs = []
    for k in range(NCHUNK):
        off = k * CATOMS
        descs.append((
            pltpu.async_copy(atoms_hbm.at[pl.ds(base + off, CATOMS)],
                             atoms_v.at[pl.ds(off, CATOMS)], sems[k]),
            pltpu.async_copy(segs_hbm.at[pl.ds(base + off, CATOMS)],
                             segs_v.at[pl.ds(off, CATOMS)], sems[k]),
        ))

    zeros = jnp.zeros((L,), jnp.float32)
    iota = lax.iota(jnp.int32, L)

    @plsc.parallel_loop(0, ROWS, unroll=8)
    def _(r):
        for k in range(128 // L):
            acc_s[r, pl.ds(k * L, L)] = zeros
            acc_c[r, pl.ds(k * L, L)] = zeros

    @plsc.parallel_loop(0, ROWS // L, unroll=4)
    def _(r):
        rows_v[pl.ds(r * L, L)] = iota + r * L

    inext = jnp.minimum(iota + 1, L - 1)
    iprev = jnp.maximum(iota - 1, 0)
    lastmask = iota == L - 1
    firstmask = iota == 0

    w_desc.wait()
    for k in range(NCHUNK):
        for d in descs[k]:
            d.wait()

        @plsc.parallel_loop(k * VPC, (k + 1) * VPC, unroll=8)
        def _(i):
            a = atoms_v[pl.ds(i * L, L)]
            s = segs_v[pl.ds(i * L, L)]
            v = plsc.load_gather(w_v, [a])
            s_next = s.at[inext].get(mode="promise_in_bounds")
            m_end = lastmask | (s != s_next)
            c = plsc.cumsum(v)
            q = jnp.where(m_end, iota, -1)
            r = plsc.cummax(q)
            rp = jnp.where(firstmask, -1,
                           r.at[iprev].get(mode="promise_in_bounds"))
            cp = jnp.where(
                rp >= 0,
                c.at[jnp.maximum(rp, 0)].get(mode="promise_in_bounds"),
                0.0)
            s_hi = jax.lax.shift_right_logical(s, 7)
            s_lo = jax.lax.bitwise_and(s, 127)
            plsc.addupdate_scatter(acc_s, [s_hi, s_lo], c - cp, mask=m_end)
            plsc.addupdate_scatter(acc_c, [s_hi, s_lo],
                                   (iota - rp).astype(jnp.float32),
                                   mask=m_end)

    @pl.when(sid == 0)
    def _():
        pltpu.sync_copy(acc_s, sh_s)
        pltpu.sync_copy(acc_c, sh_c)

    plsc.subcore_barrier()

    @pl.when(sid != 0)
    def _():
        pltpu.sync_copy(acc_s, sh_s.at[rows_v], add=True)
        pltpu.sync_copy(acc_c, sh_c.at[rows_v], add=True)

    plsc.subcore_barrier()

    pltpu.sync_copy(sh_s.at[pl.ds(sid * RPT, RPT)], bounce)
    pltpu.sync_copy(bounce, out_sum.at[cid, pl.ds(sid * RPT, RPT)])
    pltpu.sync_copy(sh_c.at[pl.ds(sid * RPT, RPT)], bounce)
    pltpu.sync_copy(bounce, out_cnt.at[cid, pl.ds(sid * RPT, RPT)])


def _combine_body(ps_ref, pc_ref, out_ref):
    s = ps_ref[0] + ps_ref[1]
    c = pc_ref[0] + pc_ref[1]
    out_ref[...] = s / jnp.maximum(c, 1.0)


def kernel(weight, atomic_numbers, segment_ids):
    w = jnp.pad(weight.reshape(-1), (0, 128 - MAX_ELEM))
    ps, pc = _sc_hist(w, atomic_numbers, segment_ids)
    out = pl.pallas_call(
        _combine_body,
        out_shape=jax.ShapeDtypeStruct((ROWS, 128), jnp.float32),
    )(ps, pc)
    return out.reshape(-1)
